# baseline (speedup 1.0000x reference)
import functools

import jax
import jax.numpy as jnp
from jax import lax
from jax.experimental import pallas as pl
from jax.experimental.pallas import tpu as pltpu
from jax.experimental.pallas import tpu_sc as plsc

_WB = 128
_NBUF = 4
_WC = 1024

_COMPILER_PARAMS = pltpu.CompilerParams(
    use_tc_tiling_on_sc=True, needs_layout_passes=False
)


def _mesh():
    return plsc.VectorSubcoreMesh(core_axis_name="c", subcore_axis_name="s")


def _make_swizzle(V, D):
    info = plsc.get_sparse_core_info()
    NC, NS, L = info.num_cores, info.num_subcores, info.num_lanes
    NW = NC * NS
    G = V * D // 128
    n_full = V // _WC
    tail = V - n_full * _WC
    n_slots = -(-n_full // NW) * NW

    @functools.partial(
        pl.kernel,
        mesh=_mesh(),
        out_type=jax.ShapeDtypeStruct((G, 128), jnp.float32),
        compiler_params=_COMPILER_PARAMS,
        scratch_types=[
            pltpu.VMEM((D, _WC + 1), jnp.float32),
            pltpu.VMEM((_WC * D // 128, 128), jnp.float32),
        ],
    )
    def ka(tabT, scr, vbuf, sbuf):
        wid = lax.axis_index("s") * NC + lax.axis_index("c")
        lane = lax.iota(jnp.int32, L)

        def do_chunk(c0, wc):
            c0 = pl.multiple_of(c0, 128)
            for g in range(D // 8):
                pltpu.sync_copy(
                    tabT.at[pl.ds(8 * g, 8), pl.ds(c0, wc)],
                    vbuf.at[pl.ds(8 * g, 8), pl.ds(0, wc)],
                )
            def inner(c16, carry):
                for ci in range(16):
                    c = c16 * 16 + ci
                    j = c0 // 4 + c16 * 4 + ci // 4
                    for h in range(2):
                        v = plsc.load_gather(
                            vbuf, [lane + h * L, jnp.full((L,), c, jnp.int32)]
                        )
                        r_st = c16 * 4 + (ci * 32 + h * L) // 128
                        cbase = (ci * 32 + h * L) % 128
                        colv = (lane + (cbase + j)) & 127
                        plsc.store_scatter(
                            sbuf,
                            [jnp.full((L,), r_st, jnp.int32), colv],
                            v,
                        )
                return carry

            lax.fori_loop(0, wc // 16, inner, 0)
            row0 = pl.multiple_of(c0 * D // 128, 32)
            pltpu.sync_copy(
                sbuf.at[pl.ds(0, wc * D // 128)],
                scr.at[pl.ds(row0, wc * D // 128)],
            )

        def body(jc, carry):
            ch = wid + NW * jc

            @pl.when(ch < n_full)
            def _():
                do_chunk(ch * _WC, _WC)

            return carry

        lax.fori_loop(0, n_slots // NW, body, 0)
        c0t = n_full * _WC
        big = tail - tail % 128
        tail_specs = []
        if big:
            tail_specs.append((c0t, big))
        if tail % 128:
            tail_specs.append((c0t + big, tail % 128))
        for i, (c0s, wcs) in enumerate(tail_specs):
            @pl.when(wid == NW - 1 - i)
            def _(c0s=c0s, wcs=wcs):
                do_chunk(c0s, wcs)

    return ka


def _make_lookup(V, D, T, B):
    info = plsc.get_sparse_core_info()
    NC, NS, L = info.num_cores, info.num_subcores, info.num_lanes
    NW = NC * NS
    NB = B // _WB
    NU = T * NB
    assert NU % NW == 0
    u_per_w = NU // NW
    assert u_per_w % _NBUF == 0
    ngrp = _WB // L

    @functools.partial(
        pl.kernel,
        mesh=_mesh(),
        out_type=jax.ShapeDtypeStruct((T, D, B), jnp.float32),
        compiler_params=_COMPILER_PARAMS,
        scratch_types=[
            pltpu.VMEM((_NBUF, 1, _WB), jnp.int32),
            pltpu.VMEM((_NBUF, _WB), jnp.int32),
            pltpu.VMEM((_WB,), jnp.int32),
            pltpu.VMEM((_NBUF, _WB, 128), jnp.float32),
            pltpu.VMEM((_NBUF, D, _WB), jnp.float32),
        ]
        + [pltpu.SemaphoreType.DMA] * (2 * _NBUF),
    )
    def kb(scr, inT, outP, gidx, moff, raw, gbuf, cbuf, *sems):
        gsems = sems[:_NBUF]
        wsems = sems[_NBUF:]
        wid = lax.axis_index("s") * NC + lax.axis_index("c")
        lane = lax.iota(jnp.int32, L)

        def unit_tb(j):
            u = wid + NW * j
            return u // NB, (u % NB) * _WB

        def load_idx(j, buf):
            t, b0 = unit_tb(j)
            pltpu.sync_copy(inT.at[t, pl.ds(b0, _WB)], raw)
            for kk in range(ngrp):
                v = raw[pl.ds(kk * L, L)]
                g = jax.lax.shift_right_logical(v, 2)
                gidx[buf, 0, pl.ds(kk * L, L)] = g
                moff[buf, pl.ds(kk * L, L)] = (v & 3) * 32 + (g & 127)

        def gather_start(buf):
            pltpu.async_copy(
                scr.at[gidx.at[buf, 0]], gbuf.at[buf], gsems[buf]
            )

        def gather_wait(buf):
            pltpu.make_async_copy(
                scr.at[gidx.at[buf, 0]], gbuf.at[buf], gsems[buf]
            ).wait()

        def extract(buf):
            for kk in range(ngrp):
                rows = lane + kk * L
                cols0 = moff[buf, pl.ds(kk * L, L)]
                for d in range(D):
                    cbuf[buf, d, pl.ds(kk * L, L)] = plsc.load_gather(
                        gbuf.at[buf], [rows, (cols0 + d) & 127]
                    )

        def write_start(j, buf):
            t, b0 = unit_tb(j)
            pltpu.async_copy(
                cbuf.at[buf], outP.at[t, :, pl.ds(b0, _WB)], wsems[buf]
            )

        def write_wait(buf):
            pltpu.make_async_copy(
                cbuf.at[buf], outP.at[0, :, pl.ds(0, _WB)], wsems[buf]
            ).wait()

        for r in range(_NBUF - 1):
            load_idx(r, r)
            gather_start(r)

        def body(jj, carry):
            for r in range(_NBUF):
                u = _NBUF * jj + r
                nxt = u + _NBUF - 1
                nbuf = (r + _NBUF - 1) % _NBUF

                @pl.when(nxt < u_per_w)
                def _():
                    load_idx(nxt, nbuf)
                    gather_start(nbuf)

                gather_wait(r)

                @pl.when(jj > 0)
                def _():
                    write_wait(r)

                extract(r)
                write_start(u, r)
            return carry

        lax.fori_loop(0, u_per_w // _NBUF, body, 0)
        for r in range(_NBUF):
            write_wait(r)

    return kb


def kernel(inputs, embedding_weight):
    Bt, T = inputs.shape
    V, D = embedding_weight.shape
    tabT = embedding_weight.T
    inT = inputs.T
    scr = _make_swizzle(V, D)(tabT)
    outP = _make_lookup(V, D, T, Bt)(scr, inT)
    return outP.transpose(2, 0, 1)
